# baseline (speedup 1.0000x reference)
import functools
import jax
import jax.numpy as jnp
from jax import lax
from jax.experimental import pallas as pl
from jax.experimental.pallas import tpu as pltpu
from jax.experimental.pallas import tpu_sc as plsc

NUM_TOKENS = 16384 * 50
DIM = 64
POS = 50
SEQ = 16384
NC, NS = 2, 16
NW = NC * NS
CHUNK = 128
UNITS = NUM_TOKENS // CHUNK
UPW = UNITS // NW
SBLK = SEQ // CHUNK
NBUF = 4
OUTER = UPW // NBUF
NUM_EMB_PAIRS = 500000


@functools.partial(
    pl.kernel,
    mesh=plsc.VectorSubcoreMesh(core_axis_name="c", subcore_axis_name="s"),
    out_type=jax.ShapeDtypeStruct((POS, DIM, SEQ), jnp.float32),
    scratch_types=(
        [pltpu.VMEM((UPW, CHUNK), jnp.int32)]
        + [pltpu.VMEM((CHUNK, 128), jnp.float32) for _ in range(NBUF)]
        + [pltpu.VMEM((DIM, CHUNK), jnp.float32) for _ in range(NBUF)]
        + [pltpu.VMEM((CHUNK,), jnp.int32) for _ in range(NBUF)]
        + [pltpu.SemaphoreType.DMA for _ in range(2 * NBUF)]
    ),
    compiler_params=pltpu.CompilerParams(use_tc_tiling_on_sc=True,
                                         needs_layout_passes=False),
)
def _embedding_gather(table_hbm, idx_hbm, out_hbm, idx_v, *scratch):
    gbuf = scratch[:NBUF]
    tbuf = scratch[NBUF:2 * NBUF]
    gidx = scratch[2 * NBUF:3 * NBUF]
    gsem = scratch[3 * NBUF:4 * NBUF]
    osem = scratch[4 * NBUF:]

    wid = lax.axis_index("s") * NC + lax.axis_index("c")
    ubase = wid * UPW

    pltpu.sync_copy(idx_hbm.at[wid], idx_v)

    lane = lax.iota(jnp.int32, 16)

    def fire_gather(jloc, b):
        for jg in range(CHUNK // 16):
            tid = idx_v[jloc, pl.ds(jg * 16, 16)]
            gidx[b][pl.ds(jg * 16, 16)] = tid >> 1
        pltpu.async_copy(table_hbm.at[gidx[b]], gbuf[b], gsem[b])

    def gather_wait(b):
        pltpu.make_async_copy(table_hbm.at[gidx[b]], gbuf[b], gsem[b]).wait()

    def transpose_unit(jloc, b):
        cols = []
        for jg in range(CHUNK // 16):
            tid = idx_v[jloc, pl.ds(jg * 16, 16)]
            cols.append((tid & 1) << 6)
        rows = [jg * 16 + lane for jg in range(CHUNK // 16)]

        def dbody(db, carry):
            base = db * 16
            for k in range(16):
                d = base + k
                for jg in range(CHUNK // 16):
                    v = plsc.load_gather(gbuf[b], [rows[jg], cols[jg] + d])
                    tbuf[b][d, pl.ds(jg * 16, 16)] = v
            return carry

        lax.fori_loop(0, DIM // 16, dbody, 0)

    def out_start(u, b):
        p = u // SBLK
        sb = u % SBLK
        pltpu.async_copy(tbuf[b], out_hbm.at[p, :, pl.ds(sb * CHUNK, CHUNK)],
                         osem[b])

    def out_wait(b):
        pltpu.make_async_copy(tbuf[b], out_hbm.at[0, :, pl.ds(0, CHUNK)],
                              osem[b]).wait()

    for b in range(NBUF):
        fire_gather(b, b)

    def body(i, carry):
        for b in range(NBUF):
            iloc = i * NBUF + b
            gather_wait(b)

            @pl.when(iloc >= NBUF)
            def _():
                out_wait(b)

            transpose_unit(iloc, b)
            out_start(ubase + iloc, b)
            nxt = iloc + NBUF

            @pl.when(nxt < UPW)
            def _():
                fire_gather(nxt, b)

        return carry

    lax.fori_loop(0, OUTER, body, 0)

    for b in range(NBUF):
        out_wait(b)


def kernel(token_ids, weight):
    idx = token_ids.T.reshape(NW, UPW, CHUNK).astype(jnp.int32)
    table = weight.reshape(NUM_EMB_PAIRS, 128)
    out = _embedding_gather(table, idx)
    return out.transpose(2, 0, 1)


# speedup vs baseline: 1.8685x; 1.8685x over previous
import functools
import jax
import jax.numpy as jnp
from jax import lax
from jax.experimental import pallas as pl
from jax.experimental.pallas import tpu as pltpu
from jax.experimental.pallas import tpu_sc as plsc

NUM_TOKENS = 16384 * 50
DIM = 64
POS = 50
SEQ = 16384
NC, NS = 2, 16
NW = NC * NS
CHUNK = 128
UNITS = NUM_TOKENS // CHUNK
UPW = UNITS // NW
SBLK = SEQ // CHUNK
NBUF = 4
OUTER = UPW // NBUF
NUM_EMB_PAIRS = 500000


@functools.partial(
    pl.kernel,
    mesh=plsc.VectorSubcoreMesh(core_axis_name="c", subcore_axis_name="s"),
    out_type=jax.ShapeDtypeStruct((POS, DIM, SEQ), jnp.float32),
    scratch_types=(
        [pltpu.VMEM((UPW, CHUNK), jnp.int32)]
        + [pltpu.VMEM((CHUNK, 128), jnp.float32) for _ in range(NBUF)]
        + [pltpu.VMEM((DIM, CHUNK), jnp.float32) for _ in range(NBUF)]
        + [pltpu.VMEM((CHUNK,), jnp.int32) for _ in range(NBUF)]
        + [pltpu.SemaphoreType.DMA for _ in range(2 * NBUF)]
    ),
    compiler_params=pltpu.CompilerParams(use_tc_tiling_on_sc=True,
                                         needs_layout_passes=False),
)
def _embedding_gather(table_hbm, idx_hbm, out_hbm, idx_v, *scratch):
    gbuf = scratch[:NBUF]
    tbuf = scratch[NBUF:2 * NBUF]
    gidx = scratch[2 * NBUF:3 * NBUF]
    gsem = scratch[3 * NBUF:4 * NBUF]
    osem = scratch[4 * NBUF:]

    wid = lax.axis_index("s") * NC + lax.axis_index("c")
    ubase = wid * UPW

    pltpu.sync_copy(idx_hbm.at[wid], idx_v)

    lane = lax.iota(jnp.int32, 16)

    def fire_gather(jloc, b):
        for jg in range(CHUNK // 16):
            tid = idx_v[jloc, pl.ds(jg * 16, 16)]
            gidx[b][pl.ds(jg * 16, 16)] = tid >> 1
        pltpu.async_copy(table_hbm.at[gidx[b]], gbuf[b], gsem[b])

    def gather_wait(b):
        pltpu.make_async_copy(table_hbm.at[gidx[b]], gbuf[b], gsem[b]).wait()

    def transpose_unit(jloc, b):
        cols = []
        for jg in range(CHUNK // 16):
            tid = idx_v[jloc, pl.ds(jg * 16, 16)]
            cols.append((tid & 1) << 6)
        rows = [jg * 16 + lane for jg in range(CHUNK // 16)]

        def dbody(db, carry):
            base = db * 16
            for s in range(16):
                t = (lane + s) & 15
                dvec = t + base
                for jg in range(CHUNK // 16):
                    v = plsc.load_gather(gbuf[b], [rows[jg], cols[jg] + dvec])
                    plsc.store_scatter(tbuf[b], [dvec, rows[jg]], v)
            return carry

        lax.fori_loop(0, DIM // 16, dbody, 0)

    def out_start(u, b):
        p = u // SBLK
        sb = u % SBLK
        pltpu.async_copy(tbuf[b], out_hbm.at[p, :, pl.ds(sb * CHUNK, CHUNK)],
                         osem[b])

    def out_wait(b):
        pltpu.make_async_copy(tbuf[b], out_hbm.at[0, :, pl.ds(0, CHUNK)],
                              osem[b]).wait()

    for b in range(NBUF):
        fire_gather(b, b)

    def body(i, carry):
        for b in range(NBUF):
            iloc = i * NBUF + b
            gather_wait(b)

            @pl.when(iloc >= NBUF)
            def _():
                out_wait(b)

            transpose_unit(iloc, b)
            out_start(ubase + iloc, b)
            nxt = iloc + NBUF

            @pl.when(nxt < UPW)
            def _():
                fire_gather(nxt, b)

        return carry

    lax.fori_loop(0, OUTER, body, 0)

    for b in range(NBUF):
        out_wait(b)


def kernel(token_ids, weight):
    idx = token_ids.T.reshape(NW, UPW, CHUNK).astype(jnp.int32)
    table = weight.reshape(NUM_EMB_PAIRS, 128)
    out = _embedding_gather(table, idx)
    return out.transpose(2, 0, 1)
